# baseline (speedup 1.0000x reference)
import functools

import jax
import jax.numpy as jnp
from jax import lax
from jax.experimental import pallas as pl
from jax.experimental.pallas import tpu as pltpu
from jax.experimental.pallas import tpu_sc as plsc

_N = 10000
_D = 128
_E = 320000
_NEG = 0.2
_COLS = 144
_ROWS = 10016
_NW = 32
_CHUNK = 64
_CPW = 162
_EPAD = _NW * _CPW * _CHUNK
_RPS = _ROWS // 16


def _proj_body(h_ref, wl_ref, bl_ref, wr_ref, br_ref, xlp_ref, xrp_ref):
    h = h_ref[...]
    xl = jnp.dot(h, wl_ref[...], preferred_element_type=jnp.float32) + bl_ref[...]
    xr = jnp.dot(h, wr_ref[...], preferred_element_type=jnp.float32) + br_ref[...]
    tail = jnp.where(
        lax.broadcasted_iota(jnp.int32, (_N, _COLS - _D), 1) == 0, 1.0, 0.0
    ).astype(jnp.float32)
    xlp_ref[...] = jnp.concatenate([xl, tail], axis=1)
    xrp_ref[...] = jnp.concatenate(
        [xr, jnp.zeros((_ROWS - _N, _D), jnp.float32)], axis=0
    )


def _proj(h, Wl, bl, Wr, br):
    return pl.pallas_call(
        _proj_body,
        out_shape=(
            jax.ShapeDtypeStruct((_N, _COLS), jnp.float32),
            jax.ShapeDtypeStruct((_ROWS, _D), jnp.float32),
        ),
    )(h, Wl, bl.reshape(1, _D), Wr, br.reshape(1, _D))


def _comb_body(v_ref, bias_ref, h_ref):
    v = v_ref[0] + v_ref[1]
    num = v[:_N, :_D]
    den = v[:_N, _D:_D + 1]
    h_ref[...] = num / (den + 1e-16) + bias_ref[...]


def _comb(vacc, bias):
    return pl.pallas_call(
        _comb_body,
        out_shape=jax.ShapeDtypeStruct((_N, _D), jnp.float32),
    )(vacc, bias.reshape(1, _D))


_mesh = plsc.VectorSubcoreMesh(core_axis_name="c", subcore_axis_name="s")


@functools.partial(
    pl.kernel,
    mesh=_mesh,
    compiler_params=pltpu.CompilerParams(
        use_tc_tiling_on_sc=False, needs_layout_passes=False),
    out_type=jax.ShapeDtypeStruct((2, _ROWS, _COLS), jnp.float32),
    scratch_types=[
        pltpu.VMEM((1, _CHUNK), jnp.int32),
        pltpu.VMEM((1, _CHUNK), jnp.int32),
        pltpu.VMEM((1, _CHUNK), jnp.int32),
        pltpu.VMEM((1, _CHUNK), jnp.int32),
        pltpu.VMEM((_CHUNK, _COLS), jnp.float32),
        pltpu.VMEM((_CHUNK, _COLS), jnp.float32),
        pltpu.VMEM((_CHUNK, _D), jnp.float32),
        pltpu.VMEM((_CHUNK, _D), jnp.float32),
        pltpu.VMEM((_D,), jnp.float32),
        pltpu.VMEM((_CHUNK,), jnp.float32),
        pltpu.VMEM_SHARED((_ROWS, _COLS), jnp.float32),
        [pltpu.SemaphoreType.DMA] * 8,
    ],
)
def _sc_edge_pass(xlp_hbm, xrp_hbm, src_hbm, dst_hbm, att_hbm, zeros_hbm,
                  out_hbm, sidx0, sidx1, didx0, didx1, xl0, xl1, xr0, xr1,
                  att_v, p_v, v_sh, sems):
    (ssi0, ssi1, sdi0, sdi1, sa0, sa1, sb0, sb1) = sems
    cid = lax.axis_index("c")
    sid = lax.axis_index("s")
    w = sid * 2 + cid
    last = _CPW - 1

    pltpu.sync_copy(att_hbm, att_v)
    pltpu.sync_copy(zeros_hbm.at[pl.ds(sid * _RPS, _RPS)],
                    v_sh.at[pl.ds(sid * _RPS, _RPS)])
    plsc.subcore_barrier()

    att_r = [att_v[pl.ds(k * 16, 16)] for k in range(_D // 16)]
    lane_iota = lax.iota(jnp.int32, 16)

    def compute(xlb, xrb):
        @plsc.parallel_loop(0, _CHUNK // 16, 1)
        def group(g):
            logits = jnp.zeros((16,), jnp.float32)
            for i in range(16):
                e = g * 16 + i
                acc = jnp.zeros((16,), jnp.float32)
                for k in range(_D // 16):
                    a = xlb[e, pl.ds(k * 16, 16)]
                    b = xrb[e, pl.ds(k * 16, 16)]
                    s = a + b
                    t = jnp.maximum(s, _NEG * s)
                    acc = acc + att_r[k] * t
                logits = jnp.where(lane_iota == i, jnp.sum(acc), logits)
            pvec = jnp.exp(logits)
            for i in range(16):
                e = g * 16 + i
                pe = pvec[i]
                for k in range(_COLS // 16):
                    xlb[e, pl.ds(k * 16, 16)] = pe * xlb[e, pl.ds(k * 16, 16)]

    def wait_idx(dst, sem):
        pltpu.make_async_copy(src_hbm.at[0, 0], dst, sem).wait()

    def wait_gather(table, idx, dst, sem):
        pltpu.make_async_copy(table.at[idx.at[0]], dst, sem).wait()

    def body_one(c, sidx_b, didx_b, xl_b, xr_b, ssi_b, sdi_b, sa_b, sb_b,
                 didx_o, xr_o, sdi_o, sb_o):
        wait_gather(xlp_hbm, sidx_b, xl_b, sa_b)
        wait_gather(xrp_hbm, didx_b, xr_b, sb_b)
        pltpu.async_copy(src_hbm.at[w, jnp.minimum(c + 2, last)], sidx_b,
                         ssi_b)
        wait_idx(didx_o, sdi_o)
        pltpu.async_copy(xrp_hbm.at[didx_o.at[0]], xr_o, sb_o)
        compute(xl_b, xr_b)
        pltpu.sync_copy(xl_b, v_sh.at[didx_b.at[0]], add=True)
        pltpu.async_copy(dst_hbm.at[w, jnp.minimum(c + 2, last)], didx_b,
                         sdi_b)
        wait_idx(sidx_b, ssi_b)
        pltpu.async_copy(xlp_hbm.at[sidx_b.at[0]], xl_b, sa_b)

    pltpu.sync_copy(src_hbm.at[w, 0], sidx0)
    pltpu.sync_copy(dst_hbm.at[w, 0], didx0)
    pltpu.sync_copy(src_hbm.at[w, 1], sidx1)
    pltpu.async_copy(xlp_hbm.at[sidx0.at[0]], xl0, sa0)
    pltpu.async_copy(xrp_hbm.at[didx0.at[0]], xr0, sb0)
    pltpu.async_copy(dst_hbm.at[w, 1], didx1, sdi1)
    pltpu.async_copy(xlp_hbm.at[sidx1.at[0]], xl1, sa1)

    def super_chunk(si, carry):
        c0 = 2 * si
        body_one(c0, sidx0, didx0, xl0, xr0, ssi0, sdi0, sa0, sb0,
                 didx1, xr1, sdi1, sb1)
        body_one(c0 + 1, sidx1, didx1, xl1, xr1, ssi1, sdi1, sa1, sb1,
                 didx0, xr0, sdi0, sb0)
        return carry

    lax.fori_loop(0, _CPW // 2, super_chunk, 0)

    wait_idx(didx1, sdi1)
    wait_gather(xlp_hbm, sidx0, xl0, sa0)
    wait_gather(xlp_hbm, sidx1, xl1, sa1)
    wait_gather(xrp_hbm, didx0, xr0, sb0)

    plsc.subcore_barrier()
    pltpu.sync_copy(v_sh.at[pl.ds(sid * _RPS, _RPS)],
                    out_hbm.at[cid, pl.ds(sid * _RPS, _RPS)])


def kernel(x, edge_index, Wl0, bl0, Wr0, br0, att0, bias0,
           Wl1, bl1, Wr1, br1, att1, bias1):
    loop = jnp.arange(_N, dtype=jnp.int32)
    src = jnp.concatenate([edge_index[0], loop])
    dst = jnp.concatenate([edge_index[1], loop])
    npad = _EPAD - (_E + _N)
    pid = jnp.arange(npad, dtype=jnp.int32)
    src_p = jnp.concatenate([src, (pid * 97) % _N]).reshape(_NW, _CPW, 1, _CHUNK)
    dst_p = jnp.concatenate([dst, _N + (pid % 16)]).reshape(_NW, _CPW, 1, _CHUNK)
    zeros = jnp.zeros((_ROWS, _COLS), jnp.float32)

    xlp0, xrp0 = _proj(x, Wl0, bl0, Wr0, br0)
    vacc0 = _sc_edge_pass(xlp0, xrp0, src_p, dst_p, att0, zeros)
    h = _comb(vacc0, bias0)
    xlp1, xrp1 = _proj(h, Wl1, bl1, Wr1, br1)
    vacc1 = _sc_edge_pass(xlp1, xrp1, src_p, dst_p, att1, zeros)
    return _comb(vacc1, bias1)


# speedup vs baseline: 1.3993x; 1.3993x over previous
import functools

import jax
import jax.numpy as jnp
from jax import lax
from jax.experimental import pallas as pl
from jax.experimental.pallas import tpu as pltpu
from jax.experimental.pallas import tpu_sc as plsc

_N = 10000
_D = 128
_E = 320000
_NEG = 0.2
_COLS = 144
_ROWS = 10016
_NW = 32
_CHUNK = 64
_CPW = 162
_EPAD = _NW * _CPW * _CHUNK
_RPS = _ROWS // 16


def _proj_body(h_ref, wl_ref, bl_ref, wr_ref, br_ref, xlp_ref, xrp_ref):
    h = h_ref[...]
    xl = jnp.dot(h, wl_ref[...], preferred_element_type=jnp.float32) + bl_ref[...]
    xr = jnp.dot(h, wr_ref[...], preferred_element_type=jnp.float32) + br_ref[...]
    tail = jnp.where(
        lax.broadcasted_iota(jnp.int32, (_N, _COLS - _D), 1) == 0, 1.0, 0.0
    ).astype(jnp.float32)
    xlp_ref[...] = jnp.concatenate([xl, tail], axis=1)
    xrp_ref[...] = jnp.concatenate(
        [xr, jnp.zeros((_ROWS - _N, _D), jnp.float32)], axis=0
    )


def _proj(h, Wl, bl, Wr, br):
    return pl.pallas_call(
        _proj_body,
        out_shape=(
            jax.ShapeDtypeStruct((_N, _COLS), jnp.float32),
            jax.ShapeDtypeStruct((_ROWS, _D), jnp.float32),
        ),
    )(h, Wl, bl.reshape(1, _D), Wr, br.reshape(1, _D))


def _comb_body(v_ref, bias_ref, h_ref):
    v = v_ref[0] + v_ref[1]
    num = v[:_N, :_D]
    den = v[:_N, _D:_D + 1]
    h_ref[...] = num / (den + 1e-16) + bias_ref[...]


def _comb(vacc, bias):
    return pl.pallas_call(
        _comb_body,
        out_shape=jax.ShapeDtypeStruct((_N, _D), jnp.float32),
    )(vacc, bias.reshape(1, _D))


_mesh = plsc.VectorSubcoreMesh(core_axis_name="c", subcore_axis_name="s")


@functools.partial(
    pl.kernel,
    mesh=_mesh,
    compiler_params=pltpu.CompilerParams(
        use_tc_tiling_on_sc=False, needs_layout_passes=False),
    out_type=jax.ShapeDtypeStruct((2, _ROWS, _COLS), jnp.float32),
    scratch_types=[
        pltpu.VMEM((1, _CHUNK), jnp.int32),
        pltpu.VMEM((1, _CHUNK), jnp.int32),
        pltpu.VMEM((1, _CHUNK), jnp.int32),
        pltpu.VMEM((1, _CHUNK), jnp.int32),
        pltpu.VMEM((1, _CHUNK), jnp.int32),
        pltpu.VMEM((1, _CHUNK), jnp.int32),
        pltpu.VMEM((_CHUNK, _COLS), jnp.float32),
        pltpu.VMEM((_CHUNK, _COLS), jnp.float32),
        pltpu.VMEM((_CHUNK, _D), jnp.float32),
        pltpu.VMEM((_CHUNK, _D), jnp.float32),
        pltpu.VMEM((_D,), jnp.float32),
        pltpu.VMEM((_CHUNK,), jnp.float32),
        pltpu.VMEM_SHARED((_ROWS, _COLS), jnp.float32),
        [pltpu.SemaphoreType.DMA] * 10,
    ],
)
def _sc_edge_pass(xlp_hbm, xrp_hbm, src_hbm, dst_hbm, att_hbm, zeros_hbm,
                  out_hbm, sidx0, sidx1, didx0, didx1, scix0, scix1,
                  xl0, xl1, xr0, xr1, att_v, p_v, v_sh, sems):
    (ssi0, ssi1, sdi0, sdi1, sa0, sa1, sb0, sb1, ssc0, ssc1) = sems
    cid = lax.axis_index("c")
    sid = lax.axis_index("s")
    w = sid * 2 + cid
    last = _CPW - 1

    pltpu.sync_copy(att_hbm, att_v)
    pltpu.sync_copy(zeros_hbm.at[pl.ds(sid * _RPS, _RPS)],
                    v_sh.at[pl.ds(sid * _RPS, _RPS)])
    plsc.subcore_barrier()

    att_r = [att_v[pl.ds(k * 16, 16)] for k in range(_D // 16)]
    lane_iota = lax.iota(jnp.int32, 16)

    def compute(xlb, xrb):
        def group(g, carry2):
            logits = jnp.zeros((16,), jnp.float32)
            for i in range(16):
                e = g * 16 + i
                acc = jnp.zeros((16,), jnp.float32)
                for k in range(_D // 16):
                    a = xlb[e, pl.ds(k * 16, 16)]
                    b = xrb[e, pl.ds(k * 16, 16)]
                    s = a + b
                    t = jnp.maximum(s, _NEG * s)
                    acc = acc + att_r[k] * t
                logits = jnp.where(lane_iota == i, jnp.sum(acc), logits)
            p_v[pl.ds(g * 16, 16)] = jnp.exp(logits)
            return carry2

        lax.fori_loop(0, _CHUNK // 16, group, 0)

        def scale_group(g, carry2):
            pvec = p_v[pl.ds(g * 16, 16)]
            for i in range(16):
                e = g * 16 + i
                pe = pvec[i]
                for k in range(_COLS // 16):
                    xlb[e, pl.ds(k * 16, 16)] = pe * xlb[e, pl.ds(k * 16, 16)]
            return carry2

        lax.fori_loop(0, _CHUNK // 16, scale_group, 0)

    def wait_idx(dst, sem):
        pltpu.make_async_copy(src_hbm.at[0, 0], dst, sem).wait()

    def wait_gather(table, idx, dst, sem):
        pltpu.make_async_copy(table.at[idx.at[0]], dst, sem).wait()

    def wait_scatter(xl_b, scix_b, sem):
        pltpu.make_async_copy(xl_b, v_sh.at[scix_b.at[0]], sem).wait()

    def copy_idx(dst_v, src_v):
        for k in range(_CHUNK // 16):
            dst_v[0, pl.ds(k * 16, 16)] = src_v[0, pl.ds(k * 16, 16)]

    def body_one(c, sidx_b, didx_b, scix_b, xl_b, xr_b,
                 ssi_b, sdi_b, sa_b, sb_b, ssc_b,
                 sidx_o, didx_o, scix_o, xl_o, xr_o,
                 ssi_o, sdi_o, sa_o, sb_o, ssc_o):
        nxt = jnp.minimum(c + 2, last)
        wait_gather(xlp_hbm, sidx_b, xl_b, sa_b)
        wait_gather(xrp_hbm, didx_b, xr_b, sb_b)
        copy_idx(scix_b, didx_b)
        pltpu.async_copy(dst_hbm.at[w, nxt], didx_b, sdi_b)
        pltpu.async_copy(src_hbm.at[w, nxt], sidx_b, ssi_b)
        wait_idx(didx_o, sdi_o)
        pltpu.async_copy(xrp_hbm.at[didx_o.at[0]], xr_o, sb_o)
        wait_scatter(xl_o, scix_o, ssc_o)
        wait_idx(sidx_o, ssi_o)
        pltpu.async_copy(xlp_hbm.at[sidx_o.at[0]], xl_o, sa_o)
        compute(xl_b, xr_b)
        pltpu.async_copy(xl_b, v_sh.at[scix_b.at[0]], ssc_b,
                         add=True)

    pltpu.sync_copy(src_hbm.at[w, 0], sidx0)
    pltpu.sync_copy(dst_hbm.at[w, 0], didx0)
    for k in range(_CHUNK // 16):
        scix1[0, pl.ds(k * 16, 16)] = _N + lane_iota
    pltpu.async_copy(xlp_hbm.at[sidx0.at[0]], xl0, sa0)
    pltpu.async_copy(xrp_hbm.at[didx0.at[0]], xr0, sb0)
    pltpu.async_copy(src_hbm.at[w, 1], sidx1, ssi1)
    pltpu.async_copy(dst_hbm.at[w, 1], didx1, sdi1)
    pltpu.async_copy(xl1, v_sh.at[scix1.at[0]], ssc1, add=True)

    def super_chunk(si, carry):
        c0 = 2 * si
        body_one(c0, sidx0, didx0, scix0, xl0, xr0,
                 ssi0, sdi0, sa0, sb0, ssc0,
                 sidx1, didx1, scix1, xl1, xr1,
                 ssi1, sdi1, sa1, sb1, ssc1)
        body_one(c0 + 1, sidx1, didx1, scix1, xl1, xr1,
                 ssi1, sdi1, sa1, sb1, ssc1,
                 sidx0, didx0, scix0, xl0, xr0,
                 ssi0, sdi0, sa0, sb0, ssc0)
        return carry

    lax.fori_loop(0, _CPW // 2, super_chunk, 0)

    wait_scatter(xl1, scix1, ssc1)
    wait_gather(xlp_hbm, sidx0, xl0, sa0)
    wait_gather(xrp_hbm, didx0, xr0, sb0)
    wait_idx(sidx1, ssi1)
    wait_idx(didx1, sdi1)

    plsc.subcore_barrier()
    pltpu.sync_copy(v_sh.at[pl.ds(sid * _RPS, _RPS)],
                    out_hbm.at[cid, pl.ds(sid * _RPS, _RPS)])


def kernel(x, edge_index, Wl0, bl0, Wr0, br0, att0, bias0,
           Wl1, bl1, Wr1, br1, att1, bias1):
    loop = jnp.arange(_N, dtype=jnp.int32)
    src = jnp.concatenate([edge_index[0], loop])
    dst = jnp.concatenate([edge_index[1], loop])
    npad = _EPAD - (_E + _N)
    pid = jnp.arange(npad, dtype=jnp.int32)
    src_p = jnp.concatenate([src, (pid * 97) % _N]).reshape(_NW, _CPW, 1, _CHUNK)
    dst_p = jnp.concatenate([dst, _N + (pid % 16)]).reshape(_NW, _CPW, 1, _CHUNK)
    zeros = jnp.zeros((_ROWS, _COLS), jnp.float32)

    xlp0, xrp0 = _proj(x, Wl0, bl0, Wr0, br0)
    vacc0 = _sc_edge_pass(xlp0, xrp0, src_p, dst_p, att0, zeros)
    h = _comb(vacc0, bias0)
    xlp1, xrp1 = _proj(h, Wl1, bl1, Wr1, br1)
    vacc1 = _sc_edge_pass(xlp1, xrp1, src_p, dst_p, att1, zeros)
    return _comb(vacc1, bias1)
